# baseline (speedup 1.0000x reference)
import functools

import jax
import jax.numpy as jnp
from jax import lax
from jax.experimental import pallas as pl
from jax.experimental.pallas import tpu as pltpu
from jax.experimental.pallas import tpu_sc as plsc

_TBLK = 2048


def _table_rows_main(vocab, dim, n_main):
    def body(tt_ref, out_ref):
        out_ref[:, 0:dim] = jnp.transpose(tt_ref[...], (1, 0))

    return pl.pallas_call(
        body,
        grid=(n_main // _TBLK,),
        in_specs=[pl.BlockSpec((dim, _TBLK), lambda i: (0, i))],
        out_specs=pl.BlockSpec((_TBLK, 128), lambda i: (i, 0)),
        out_shape=jax.ShapeDtypeStruct((vocab + 64, 128), jnp.float32),
    )


def _table_rows_tail(vocab, dim, n_main):
    def body(big_ref, tail_ref, out_ref):
        del big_ref
        out_ref[:, 0:dim] = jnp.transpose(tail_ref[...], (1, 0))

    n_tail = vocab + 64 - n_main
    return pl.pallas_call(
        body,
        grid=(n_tail // 128,),
        in_specs=[
            pl.BlockSpec(memory_space=pl.ANY),
            pl.BlockSpec((dim, 128), lambda i: (0, i)),
        ],
        out_specs=pl.BlockSpec((128, 128), lambda i: (n_main // 128 + i, 0)),
        out_shape=jax.ShapeDtypeStruct((vocab + 64, 128), jnp.float32),
        input_output_aliases={0: 0},
    )


def _emb_gather(n_total, n_rows, dim):
    info = plsc.get_sparse_core_info()
    nw = info.num_cores * info.num_subcores
    n_per_w = n_total // nw
    chunk = 1600
    n_chunks = n_per_w // chunk
    assert n_per_w % chunk == 0 and n_chunks >= 4

    mesh = plsc.VectorSubcoreMesh(core_axis_name="c", subcore_axis_name="s")

    @functools.partial(
        pl.kernel,
        mesh=mesh,
        compiler_params=pltpu.CompilerParams(use_tc_tiling_on_sc=False),
        out_type=jax.ShapeDtypeStruct((n_total, dim), jnp.float32),
        scratch_types=[
            pltpu.VMEM((2, chunk), jnp.int32),
            pltpu.VMEM((2, chunk, dim), jnp.float32),
            pltpu.SemaphoreType.DMA,
            pltpu.SemaphoreType.DMA,
            pltpu.SemaphoreType.DMA,
            pltpu.SemaphoreType.DMA,
            pltpu.SemaphoreType.DMA,
            pltpu.SemaphoreType.DMA,
        ],
    )
    def emb(idx_hbm, table_hbm, out_hbm, idx_v, rows_v,
            semi0, semi1, semg0, semg1, semo0, semo1):
        semi = (semi0, semi1)
        semg = (semg0, semg1)
        semo = (semo0, semo1)
        wid = lax.axis_index("s") * info.num_cores + lax.axis_index("c")
        base = wid * n_per_w

        idx_cp = [None] * n_chunks
        g_cp = [None] * n_chunks
        o_cp = [None] * n_chunks

        def issue_idx(i):
            off = base + i * chunk
            cp = pltpu.make_async_copy(
                idx_hbm.at[pl.ds(off, chunk)], idx_v.at[i % 2], semi[i % 2])
            cp.start()
            idx_cp[i] = cp

        def issue_gather(i):
            cp = pltpu.make_async_copy(
                table_hbm.at[idx_v.at[i % 2]], rows_v.at[i % 2], semg[i % 2])
            cp.start()
            g_cp[i] = cp

        def issue_out(i):
            off = base + i * chunk
            cp = pltpu.make_async_copy(
                rows_v.at[i % 2], out_hbm.at[pl.ds(off, chunk)], semo[i % 2])
            cp.start()
            o_cp[i] = cp

        issue_idx(0)
        issue_idx(1)
        idx_cp[0].wait()
        issue_gather(0)
        for i in range(n_chunks):
            g_cp[i].wait()
            issue_out(i)
            if i + 2 < n_chunks:
                issue_idx(i + 2)
            if i + 1 < n_chunks:
                idx_cp[i + 1].wait()
                if i >= 1:
                    o_cp[i - 1].wait()
                issue_gather(i + 1)
        o_cp[n_chunks - 2].wait()
        o_cp[n_chunks - 1].wait()

    return emb


def _finish(batch, seq, dim):
    kb = 16
    nst = seq // 4
    sg = 128 // dim

    def body(a_ref, o_ref):
        blk = a_ref[...].reshape(kb, nst, 8, 128)
        for st in range(nst):
            for sr in range(sg):
                piece = blk[:, st, :, sr * dim:(sr + 1) * dim]
                piece = piece.reshape(kb * 8, dim)
                o_ref[st * sg + sr, :, :] = jnp.transpose(piece, (1, 0))

    return pl.pallas_call(
        body,
        grid=(batch // (8 * kb),),
        in_specs=[pl.BlockSpec((kb * nst * 8, 128), lambda i: (i, 0))],
        out_specs=pl.BlockSpec((seq, dim, 8 * kb), lambda i: (0, 0, i)),
        out_shape=jax.ShapeDtypeStruct((seq, dim, batch), jnp.float32),
    )


def kernel(x, token_emb_table):
    batch, seq = x.shape
    vocab, dim = token_emb_table.shape
    n_total = batch * seq
    n_main = (vocab // _TBLK) * _TBLK

    tt = token_emb_table.T
    rows_main = _table_rows_main(vocab, dim, n_main)(tt)
    tail = jnp.pad(tt[:, n_main:], ((0, 0), (0, 64 - (vocab - n_main) % 64)))
    rows128 = _table_rows_tail(vocab, dim, n_main)(rows_main, tail)
    table_lin = rows128.reshape((vocab + 64) * (128 // dim), dim)

    xg = (x.reshape(batch // 8, 8, seq // 4, 4)
          .transpose(0, 2, 1, 3).reshape(n_total) * (128 // dim))
    xg = xg.astype(jnp.int32)

    emb = _emb_gather(n_total, (vocab + 64) * (128 // dim), dim)
    flat = emb(xg, table_lin)

    a2d = flat.reshape(n_total * dim // 128, 128)
    o3 = _finish(batch, seq, dim)(a2d)
    return o3.transpose(2, 0, 1)


# speedup vs baseline: 1.3891x; 1.3891x over previous
import functools

import jax
import jax.numpy as jnp
from jax import lax
from jax.experimental import pallas as pl
from jax.experimental.pallas import tpu as pltpu
from jax.experimental.pallas import tpu_sc as plsc

_TBLK = 2048


_NROW = 250368


def _table_rows_main(vocab, dim, n_main):
    sub = _TBLK // 4

    def body(t0, t1, t2, t3, out_ref):
        parts = [jnp.transpose(t[...], (1, 0)) for t in (t0, t1, t2, t3)]
        out_ref[...] = jnp.concatenate(parts, axis=1)

    def mk_in(a):
        return pl.BlockSpec((dim, sub), lambda i, a=a: (0, 4 * i + a))

    return pl.pallas_call(
        body,
        grid=(n_main // _TBLK,),
        in_specs=[mk_in(a) for a in range(4)],
        out_specs=pl.BlockSpec((sub, 128), lambda i: (i, 0)),
        out_shape=jax.ShapeDtypeStruct((_NROW, 128), jnp.float32),
    )


def _table_rows_tail(vocab, dim, n_main):
    sub = _TBLK // 4
    base = n_main // _TBLK

    def body(big_ref, t0, t1, t2, t3, out_ref):
        del big_ref
        parts = [jnp.transpose(t[...], (1, 0)) for t in (t0, t1, t2, t3)]
        out_ref[...] = jnp.concatenate(parts, axis=1)

    def mk_in(a):
        return pl.BlockSpec((dim, sub), lambda i, a=a: (0, a))

    return pl.pallas_call(
        body,
        grid=(1,),
        in_specs=[pl.BlockSpec(memory_space=pl.ANY)] + [mk_in(a)
                                                        for a in range(4)],
        out_specs=pl.BlockSpec((sub, 128), lambda i: (base + i, 0)),
        out_shape=jax.ShapeDtypeStruct((_NROW, 128), jnp.float32),
        input_output_aliases={0: 0},
    )


def _emb_gather(n_total, n_rows, dim):
    info = plsc.get_sparse_core_info()
    nw = info.num_cores * info.num_subcores
    n_per_w = n_total // nw
    chunk = 1600
    n_chunks = n_per_w // chunk
    assert n_per_w % chunk == 0 and n_chunks >= 4

    mesh = plsc.VectorSubcoreMesh(core_axis_name="c", subcore_axis_name="s")

    @functools.partial(
        pl.kernel,
        mesh=mesh,
        compiler_params=pltpu.CompilerParams(use_tc_tiling_on_sc=False),
        out_type=jax.ShapeDtypeStruct((n_total, dim), jnp.float32),
        scratch_types=[
            pltpu.VMEM((2, chunk), jnp.int32),
            pltpu.VMEM((2, chunk, dim), jnp.float32),
            pltpu.SemaphoreType.DMA,
            pltpu.SemaphoreType.DMA,
            pltpu.SemaphoreType.DMA,
            pltpu.SemaphoreType.DMA,
            pltpu.SemaphoreType.DMA,
            pltpu.SemaphoreType.DMA,
        ],
    )
    def emb(idx_hbm, table_hbm, out_hbm, idx_v, rows_v,
            semi0, semi1, semg0, semg1, semo0, semo1):
        semi = (semi0, semi1)
        semg = (semg0, semg1)
        semo = (semo0, semo1)
        wid = lax.axis_index("s") * info.num_cores + lax.axis_index("c")
        base = wid * n_per_w

        idx_cp = [None] * n_chunks
        g_cp = [None] * n_chunks
        o_cp = [None] * n_chunks

        def issue_idx(i):
            off = base + i * chunk
            cp = pltpu.make_async_copy(
                idx_hbm.at[pl.ds(off, chunk)], idx_v.at[i % 2], semi[i % 2])
            cp.start()
            idx_cp[i] = cp

        def issue_gather(i):
            cp = pltpu.make_async_copy(
                table_hbm.at[idx_v.at[i % 2]], rows_v.at[i % 2], semg[i % 2])
            cp.start()
            g_cp[i] = cp

        def issue_out(i):
            off = base + i * chunk
            cp = pltpu.make_async_copy(
                rows_v.at[i % 2], out_hbm.at[pl.ds(off, chunk)], semo[i % 2])
            cp.start()
            o_cp[i] = cp

        issue_idx(0)
        issue_idx(1)
        idx_cp[0].wait()
        issue_gather(0)
        for i in range(n_chunks):
            g_cp[i].wait()
            issue_out(i)
            if i + 2 < n_chunks:
                issue_idx(i + 2)
            if i + 1 < n_chunks:
                idx_cp[i + 1].wait()
                if i >= 1:
                    o_cp[i - 1].wait()
                issue_gather(i + 1)
        o_cp[n_chunks - 2].wait()
        o_cp[n_chunks - 1].wait()

    return emb


def _finish(batch, seq, dim):
    kb = 16
    nst = seq // 4
    sg = 128 // dim

    def body(a_ref, o_ref):
        blk = a_ref[...].reshape(kb, 8, nst, 128)
        for st in range(nst):
            for sr in range(sg):
                piece = blk[:, :, st, sr * dim:(sr + 1) * dim]
                piece = piece.reshape(kb * 8, dim)
                o_ref[st * sg + sr, :, :] = jnp.transpose(piece, (1, 0))

    return pl.pallas_call(
        body,
        grid=(batch // (8 * kb),),
        in_specs=[pl.BlockSpec((kb * nst * 8, 128), lambda i: (i, 0))],
        out_specs=pl.BlockSpec((seq, dim, 8 * kb), lambda i: (0, 0, i)),
        out_shape=jax.ShapeDtypeStruct((seq, dim, batch), jnp.float32),
    )


def kernel(x, token_emb_table):
    batch, seq = x.shape
    vocab, dim = token_emb_table.shape
    n_total = batch * seq
    n_main = (vocab // _TBLK) * _TBLK

    tt = token_emb_table.T
    rows_main = _table_rows_main(vocab, dim, n_main)(tt, tt, tt, tt)
    tail = jnp.pad(tt[:, n_main:], ((0, 0), (0, _TBLK - (vocab - n_main))))
    rows128 = _table_rows_tail(vocab, dim, n_main)(
        rows_main, tail, tail, tail, tail)
    table_lin = rows128.reshape(_NROW * (128 // dim), dim)

    t = x.reshape(n_total).astype(jnp.int32)
    xg = ((t >> 11) << 11) + ((t & 511) << 2) + ((t >> 9) & 3)

    emb = _emb_gather(n_total, _NROW * (128 // dim), dim)
    flat = emb(xg, table_lin)

    a2d = flat.reshape(n_total * dim // 128, 128)
    o3 = _finish(batch, seq, dim)(a2d)
    return o3.transpose(2, 0, 1)


# speedup vs baseline: 1.7690x; 1.2735x over previous
import functools

import jax
import jax.numpy as jnp
from jax import lax
from jax.experimental import pallas as pl
from jax.experimental.pallas import tpu as pltpu
from jax.experimental.pallas import tpu_sc as plsc

_TBLK = 8192


_NROW = 251904


def _table_rows_main(vocab, dim, n_main):
    sub = _TBLK // 4

    def body(t0, t1, t2, t3, out_ref):
        parts = [jnp.transpose(t[...], (1, 0)) for t in (t0, t1, t2, t3)]
        out_ref[...] = jnp.concatenate(parts, axis=1)

    def mk_in(a):
        return pl.BlockSpec((dim, sub), lambda i, a=a: (0, 4 * i + a))

    return pl.pallas_call(
        body,
        grid=(n_main // _TBLK,),
        in_specs=[mk_in(a) for a in range(4)],
        out_specs=pl.BlockSpec((sub, 128), lambda i: (i, 0)),
        out_shape=jax.ShapeDtypeStruct((_NROW, 128), jnp.float32),
    )


def _table_rows_tail(vocab, dim, n_main):
    sub = _TBLK // 4
    base = n_main // _TBLK

    def body(big_ref, t0, t1, t2, t3, out_ref):
        del big_ref
        parts = [jnp.transpose(t[...], (1, 0)) for t in (t0, t1, t2, t3)]
        out_ref[...] = jnp.concatenate(parts, axis=1)

    def mk_in(a):
        return pl.BlockSpec((dim, sub), lambda i, a=a: (0, a))

    return pl.pallas_call(
        body,
        grid=(1,),
        in_specs=[pl.BlockSpec(memory_space=pl.ANY)] + [mk_in(a)
                                                        for a in range(4)],
        out_specs=pl.BlockSpec((sub, 128), lambda i: (base + i, 0)),
        out_shape=jax.ShapeDtypeStruct((_NROW, 128), jnp.float32),
        input_output_aliases={0: 0},
    )


def _emb_gather(n_total, n_rows, dim):
    info = plsc.get_sparse_core_info()
    nw = info.num_cores * info.num_subcores
    n_per_w = n_total // nw
    chunk = 1600
    n_chunks = n_per_w // chunk
    assert n_per_w % chunk == 0 and n_chunks >= 4

    mesh = plsc.VectorSubcoreMesh(core_axis_name="c", subcore_axis_name="s")

    @functools.partial(
        pl.kernel,
        mesh=mesh,
        compiler_params=pltpu.CompilerParams(use_tc_tiling_on_sc=False),
        out_type=jax.ShapeDtypeStruct((n_total, dim), jnp.float32),
        scratch_types=[
            pltpu.VMEM((2, chunk), jnp.int32),
            pltpu.VMEM((2, chunk, dim), jnp.float32),
            pltpu.SemaphoreType.DMA,
            pltpu.SemaphoreType.DMA,
            pltpu.SemaphoreType.DMA,
            pltpu.SemaphoreType.DMA,
            pltpu.SemaphoreType.DMA,
            pltpu.SemaphoreType.DMA,
        ],
    )
    def emb(idx_hbm, table_hbm, out_hbm, idx_v, rows_v,
            semi0, semi1, semg0, semg1, semo0, semo1):
        semi = (semi0, semi1)
        semg = (semg0, semg1)
        semo = (semo0, semo1)
        wid = lax.axis_index("s") * info.num_cores + lax.axis_index("c")
        base = wid * n_per_w

        idx_cp = [None] * n_chunks
        g_cp = [None] * n_chunks
        o_cp = [None] * n_chunks

        def issue_idx(i):
            off = base + i * chunk
            cp = pltpu.make_async_copy(
                idx_hbm.at[pl.ds(off, chunk)], idx_v.at[i % 2], semi[i % 2])
            cp.start()
            idx_cp[i] = cp

        def issue_gather(i):
            cp = pltpu.make_async_copy(
                table_hbm.at[idx_v.at[i % 2]], rows_v.at[i % 2], semg[i % 2])
            cp.start()
            g_cp[i] = cp

        def issue_out(i):
            off = base + i * chunk
            cp = pltpu.make_async_copy(
                rows_v.at[i % 2], out_hbm.at[pl.ds(off, chunk)], semo[i % 2])
            cp.start()
            o_cp[i] = cp

        issue_idx(0)
        issue_idx(1)
        idx_cp[0].wait()
        issue_gather(0)
        for i in range(n_chunks):
            g_cp[i].wait()
            issue_out(i)
            if i + 2 < n_chunks:
                issue_idx(i + 2)
            if i + 1 < n_chunks:
                idx_cp[i + 1].wait()
                if i >= 1:
                    o_cp[i - 1].wait()
                issue_gather(i + 1)
        o_cp[n_chunks - 2].wait()
        o_cp[n_chunks - 1].wait()

    return emb


def _finish(batch, seq, dim):
    kb = 16
    nst = seq // 4
    sg = 128 // dim

    def body(a_ref, o_ref):
        blk = a_ref[...].reshape(kb, 8, nst, 128)
        for st in range(nst):
            for sr in range(sg):
                piece = blk[:, :, st, sr * dim:(sr + 1) * dim]
                piece = piece.reshape(kb * 8, dim)
                o_ref[st * sg + sr, :, :] = jnp.transpose(piece, (1, 0))

    return pl.pallas_call(
        body,
        grid=(batch // (8 * kb),),
        in_specs=[pl.BlockSpec((kb * nst * 8, 128), lambda i: (i, 0))],
        out_specs=pl.BlockSpec((seq, dim, 8 * kb), lambda i: (0, 0, i)),
        out_shape=jax.ShapeDtypeStruct((seq, dim, batch), jnp.float32),
    )


def kernel(x, token_emb_table):
    batch, seq = x.shape
    vocab, dim = token_emb_table.shape
    n_total = batch * seq
    n_main = (vocab // _TBLK) * _TBLK

    tt = token_emb_table.T
    rows_main = _table_rows_main(vocab, dim, n_main)(tt, tt, tt, tt)
    tail = jnp.pad(tt[:, n_main:], ((0, 0), (0, _TBLK - (vocab - n_main))))
    rows128 = _table_rows_tail(vocab, dim, n_main)(
        rows_main, tail, tail, tail, tail)
    table_lin = rows128.reshape(_NROW * (128 // dim), dim)

    t = x.reshape(n_total).astype(jnp.int32)
    xg = ((t >> 13) << 13) + ((t & 2047) << 2) + ((t >> 11) & 3)

    emb = _emb_gather(n_total, _NROW * (128 // dim), dim)
    flat = emb(xg, table_lin)

    a2d = flat.reshape(n_total * dim // 128, 128)
    o3 = _finish(batch, seq, dim)(a2d)
    return o3.transpose(2, 0, 1)


# speedup vs baseline: 1.7867x; 1.0100x over previous
import functools

import jax
import jax.numpy as jnp
from jax import lax
from jax.experimental import pallas as pl
from jax.experimental.pallas import tpu as pltpu
from jax.experimental.pallas import tpu_sc as plsc

_TBLK = 16384


_NROW = 253952


def _table_rows_main(vocab, dim, n_main):
    sub = _TBLK // 4

    def body(t0, t1, t2, t3, out_ref):
        parts = [jnp.transpose(t[...], (1, 0)) for t in (t0, t1, t2, t3)]
        out_ref[...] = jnp.concatenate(parts, axis=1)

    def mk_in(a):
        return pl.BlockSpec((dim, sub), lambda i, a=a: (0, 4 * i + a))

    return pl.pallas_call(
        body,
        grid=(n_main // _TBLK,),
        in_specs=[mk_in(a) for a in range(4)],
        out_specs=pl.BlockSpec((sub, 128), lambda i: (i, 0)),
        out_shape=jax.ShapeDtypeStruct((_NROW, 128), jnp.float32),
    )


def _table_rows_tail(vocab, dim, n_main):
    sub = _TBLK // 4
    base = n_main // _TBLK

    def body(big_ref, t0, t1, t2, t3, out_ref):
        del big_ref
        parts = [jnp.transpose(t[...], (1, 0)) for t in (t0, t1, t2, t3)]
        out_ref[...] = jnp.concatenate(parts, axis=1)

    def mk_in(a):
        return pl.BlockSpec((dim, sub), lambda i, a=a: (0, a))

    return pl.pallas_call(
        body,
        grid=(1,),
        in_specs=[pl.BlockSpec(memory_space=pl.ANY)] + [mk_in(a)
                                                        for a in range(4)],
        out_specs=pl.BlockSpec((sub, 128), lambda i: (base + i, 0)),
        out_shape=jax.ShapeDtypeStruct((_NROW, 128), jnp.float32),
        input_output_aliases={0: 0},
    )


def _emb_gather(n_total, n_rows, dim):
    info = plsc.get_sparse_core_info()
    nw = info.num_cores * info.num_subcores
    n_per_w = n_total // nw
    chunk = 1600
    n_chunks = n_per_w // chunk
    assert n_per_w % chunk == 0 and n_chunks >= 4

    mesh = plsc.VectorSubcoreMesh(core_axis_name="c", subcore_axis_name="s")

    @functools.partial(
        pl.kernel,
        mesh=mesh,
        compiler_params=pltpu.CompilerParams(use_tc_tiling_on_sc=False),
        out_type=jax.ShapeDtypeStruct((n_total, dim), jnp.float32),
        scratch_types=[
            pltpu.VMEM((2, chunk), jnp.int32),
            pltpu.VMEM((2, chunk, dim), jnp.float32),
            pltpu.SemaphoreType.DMA,
            pltpu.SemaphoreType.DMA,
            pltpu.SemaphoreType.DMA,
            pltpu.SemaphoreType.DMA,
            pltpu.SemaphoreType.DMA,
            pltpu.SemaphoreType.DMA,
        ],
    )
    def emb(idx_hbm, table_hbm, out_hbm, idx_v, rows_v,
            semi0, semi1, semg0, semg1, semo0, semo1):
        semi = (semi0, semi1)
        semg = (semg0, semg1)
        semo = (semo0, semo1)
        wid = lax.axis_index("s") * info.num_cores + lax.axis_index("c")
        base = wid * n_per_w

        idx_cp = [None] * n_chunks
        g_cp = [None] * n_chunks
        o_cp = [None] * n_chunks

        def issue_idx(i):
            off = base + i * chunk
            cp = pltpu.make_async_copy(
                idx_hbm.at[pl.ds(off, chunk)], idx_v.at[i % 2], semi[i % 2])
            cp.start()
            idx_cp[i] = cp

        def issue_gather(i):
            cp = pltpu.make_async_copy(
                table_hbm.at[idx_v.at[i % 2]], rows_v.at[i % 2], semg[i % 2])
            cp.start()
            g_cp[i] = cp

        def issue_out(i):
            off = base + i * chunk
            cp = pltpu.make_async_copy(
                rows_v.at[i % 2], out_hbm.at[pl.ds(off, chunk)], semo[i % 2])
            cp.start()
            o_cp[i] = cp

        issue_idx(0)
        issue_idx(1)
        idx_cp[0].wait()
        issue_gather(0)
        for i in range(n_chunks):
            g_cp[i].wait()
            issue_out(i)
            if i + 2 < n_chunks:
                issue_idx(i + 2)
            if i + 1 < n_chunks:
                idx_cp[i + 1].wait()
                if i >= 1:
                    o_cp[i - 1].wait()
                issue_gather(i + 1)
        o_cp[n_chunks - 2].wait()
        o_cp[n_chunks - 1].wait()

    return emb


def _finish(batch, seq, dim):
    kb = 16
    nst = seq // 4
    sg = 128 // dim

    def body(a_ref, o_ref):
        blk = a_ref[...].reshape(kb, 8, nst, 128)
        for st in range(nst):
            for sr in range(sg):
                piece = blk[:, :, st, sr * dim:(sr + 1) * dim]
                piece = piece.reshape(kb * 8, dim)
                o_ref[st * sg + sr, :, :] = jnp.transpose(piece, (1, 0))

    return pl.pallas_call(
        body,
        grid=(batch // (8 * kb),),
        in_specs=[pl.BlockSpec((kb * nst * 8, 128), lambda i: (i, 0))],
        out_specs=pl.BlockSpec((seq, dim, 8 * kb), lambda i: (0, 0, i)),
        out_shape=jax.ShapeDtypeStruct((seq, dim, batch), jnp.float32),
    )


def kernel(x, token_emb_table):
    batch, seq = x.shape
    vocab, dim = token_emb_table.shape
    n_total = batch * seq
    n_main = (vocab // _TBLK) * _TBLK

    tt = token_emb_table.T
    rows_main = _table_rows_main(vocab, dim, n_main)(tt, tt, tt, tt)
    tail = jnp.pad(tt[:, n_main:], ((0, 0), (0, _TBLK - (vocab - n_main))))
    rows128 = _table_rows_tail(vocab, dim, n_main)(
        rows_main, tail, tail, tail, tail)
    table_lin = rows128.reshape(_NROW * (128 // dim), dim)

    t = x.reshape(n_total).astype(jnp.int32)
    xg = ((t >> 14) << 14) + ((t & 4095) << 2) + ((t >> 12) & 3)

    emb = _emb_gather(n_total, _NROW * (128 // dim), dim)
    flat = emb(xg, table_lin)

    a2d = flat.reshape(n_total * dim // 128, 128)
    o3 = _finish(batch, seq, dim)(a2d)
    return o3.transpose(2, 0, 1)


# speedup vs baseline: 1.9309x; 1.0807x over previous
import functools

import jax
import jax.numpy as jnp
from jax import lax
from jax.experimental import pallas as pl
from jax.experimental.pallas import tpu as pltpu
from jax.experimental.pallas import tpu_sc as plsc

_TBLK = 16384


_NROW = 253952


def _table_rows_main(vocab, dim, n_main):
    sub = _TBLK // 4

    def body(tin, out_ref):
        for c in range(sub // 256):
            parts = [jnp.transpose(tin[:, a * sub + c * 256:
                                       a * sub + c * 256 + 256], (1, 0))
                     for a in range(4)]
            out_ref[pl.ds(c * 256, 256), :] = jnp.concatenate(parts, axis=1)

    return pl.pallas_call(
        body,
        grid=(n_main // _TBLK,),
        in_specs=[pl.BlockSpec((dim, _TBLK), lambda i: (0, i))],
        out_specs=pl.BlockSpec((sub, 128), lambda i: (i, 0)),
        out_shape=jax.ShapeDtypeStruct((_NROW, 128), jnp.float32),
    )


def _table_rows_tail(vocab, dim, n_main):
    sub = _TBLK // 4
    base = n_main // _TBLK

    def body(big_ref, t0, t1, t2, t3, out_ref):
        del big_ref
        parts = [jnp.transpose(t[...], (1, 0)) for t in (t0, t1, t2, t3)]
        out_ref[...] = jnp.concatenate(parts, axis=1)

    def mk_in(a):
        return pl.BlockSpec((dim, sub), lambda i, a=a: (0, a))

    return pl.pallas_call(
        body,
        grid=(1,),
        in_specs=[pl.BlockSpec(memory_space=pl.ANY)] + [mk_in(a)
                                                        for a in range(4)],
        out_specs=pl.BlockSpec((sub, 128), lambda i: (base + i, 0)),
        out_shape=jax.ShapeDtypeStruct((_NROW, 128), jnp.float32),
        input_output_aliases={0: 0},
    )


def _emb_gather(n_total, n_rows, dim):
    info = plsc.get_sparse_core_info()
    nw = info.num_cores * info.num_subcores
    n_per_w = n_total // nw
    chunk = 1600
    n_chunks = n_per_w // chunk
    assert n_per_w % chunk == 0 and n_chunks >= 4

    mesh = plsc.VectorSubcoreMesh(core_axis_name="c", subcore_axis_name="s")

    @functools.partial(
        pl.kernel,
        mesh=mesh,
        compiler_params=pltpu.CompilerParams(use_tc_tiling_on_sc=False),
        out_type=jax.ShapeDtypeStruct((n_total, dim), jnp.float32),
        scratch_types=[
            pltpu.VMEM((2, chunk), jnp.int32),
            pltpu.VMEM((2, chunk, dim), jnp.float32),
            pltpu.SemaphoreType.DMA,
            pltpu.SemaphoreType.DMA,
            pltpu.SemaphoreType.DMA,
            pltpu.SemaphoreType.DMA,
            pltpu.SemaphoreType.DMA,
            pltpu.SemaphoreType.DMA,
        ],
    )
    def emb(idx_hbm, table_hbm, out_hbm, idx_v, rows_v,
            semi0, semi1, semg0, semg1, semo0, semo1):
        semi = (semi0, semi1)
        semg = (semg0, semg1)
        semo = (semo0, semo1)
        wid = lax.axis_index("s") * info.num_cores + lax.axis_index("c")
        base = wid * n_per_w

        idx_cp = [None] * n_chunks
        g_cp = [None] * n_chunks
        o_cp = [None] * n_chunks

        def issue_idx(i):
            off = base + i * chunk
            cp = pltpu.make_async_copy(
                idx_hbm.at[pl.ds(off, chunk)], idx_v.at[i % 2], semi[i % 2])
            cp.start()
            idx_cp[i] = cp

        def issue_gather(i):
            cp = pltpu.make_async_copy(
                table_hbm.at[idx_v.at[i % 2]], rows_v.at[i % 2], semg[i % 2])
            cp.start()
            g_cp[i] = cp

        def issue_out(i):
            off = base + i * chunk
            cp = pltpu.make_async_copy(
                rows_v.at[i % 2], out_hbm.at[pl.ds(off, chunk)], semo[i % 2])
            cp.start()
            o_cp[i] = cp

        issue_idx(0)
        issue_idx(1)
        idx_cp[0].wait()
        issue_gather(0)
        for i in range(n_chunks):
            g_cp[i].wait()
            issue_out(i)
            if i + 2 < n_chunks:
                issue_idx(i + 2)
            if i + 1 < n_chunks:
                idx_cp[i + 1].wait()
                if i >= 1:
                    o_cp[i - 1].wait()
                issue_gather(i + 1)
        o_cp[n_chunks - 2].wait()
        o_cp[n_chunks - 1].wait()

    return emb


def _finish_part(batch, seq, dim, nparts, part):
    kb = 16
    nst = seq // 4
    sg = 128 // dim
    nblk = batch // (8 * kb * nparts)
    base = part * nblk

    def tbody(a_ref, o_ref):
        blk = a_ref[...].reshape(kb, 8, nst, 128)
        for st in range(nst):
            for sr in range(sg):
                piece = blk[:, :, st, sr * dim:(sr + 1) * dim]
                piece = piece.reshape(kb * 8, dim)
                o_ref[st * sg + sr, :, :] = jnp.transpose(piece, (1, 0))

    def body_first(a_ref, o_ref):
        tbody(a_ref, o_ref)

    def body_alias(prev_ref, a_ref, o_ref):
        del prev_ref
        tbody(a_ref, o_ref)

    nrow_half = batch * seq * dim // (128 * nparts)
    in_specs = [pl.BlockSpec((kb * nst * 8, 128), lambda i: (i, 0))]
    body = body_first
    aliases = {}
    if part > 0:
        in_specs = [pl.BlockSpec(memory_space=pl.ANY)] + in_specs
        body = body_alias
        aliases = {0: 0}
    return pl.pallas_call(
        body,
        grid=(nblk,),
        in_specs=in_specs,
        out_specs=pl.BlockSpec((seq, dim, 8 * kb), lambda i: (0, 0, base + i)),
        out_shape=jax.ShapeDtypeStruct((seq, dim, batch), jnp.float32),
        input_output_aliases=aliases,
    )


def kernel(x, token_emb_table):
    batch, seq = x.shape
    vocab, dim = token_emb_table.shape
    n_total = batch * seq
    n_main = (vocab // _TBLK) * _TBLK

    tt = token_emb_table.T
    rows_main = _table_rows_main(vocab, dim, n_main)(tt)
    tail = jnp.pad(tt[:, n_main:], ((0, 0), (0, _TBLK - (vocab - n_main))))
    rows128 = _table_rows_tail(vocab, dim, n_main)(
        rows_main, tail, tail, tail, tail)
    table_lin = rows128.reshape(_NROW * (128 // dim), dim)

    t = x.reshape(n_total).astype(jnp.int32)
    xg = ((t >> 14) << 14) + ((t & 4095) << 2) + ((t >> 12) & 3)

    nh = n_total // 2
    emb = _emb_gather(nh, _NROW * (128 // dim), dim)
    flat0 = emb(xg[:nh], table_lin)
    flat1 = emb(xg[nh:], table_lin)
    a0 = flat0.reshape(nh * dim // 128, 128)
    a1 = flat1.reshape(nh * dim // 128, 128)
    o3 = _finish_part(batch, seq, dim, 2, 0)(a0)
    o3 = _finish_part(batch, seq, dim, 2, 1)(o3, a1)
    return o3.transpose(2, 0, 1)
